# baseline (speedup 1.0000x reference)
import functools

import jax
import jax.numpy as jnp
from jax import lax
from jax.experimental import pallas as pl
from jax.experimental.pallas import tpu as pltpu
from jax.experimental.pallas import tpu_sc as plsc

N_ROWS = 100000
D = 128
NSEG = 1024
NC = 2
NS = 16
NW = NC * NS
B = 128
L = 16
NBUF = 4
FULL_BLOCKS = N_ROWS // B
TAIL = N_ROWS - FULL_BLOCKS * B
ITERS = (FULL_BLOCKS + NW - 1) // NW
ITP = 32
SEG_PER_TILE = NSEG // NS
HROWS = 16


def _sc_partials(node_feat, ids2d, seg_ids, zsum):
    mesh = plsc.VectorSubcoreMesh(core_axis_name="c", subcore_axis_name="s",
                                  num_cores=NC, num_subcores=NS)

    @functools.partial(
        pl.kernel,
        out_type=(
            jax.ShapeDtypeStruct((NC, NSEG, D), jnp.float32),
            jax.ShapeDtypeStruct((NC, HROWS, D), jnp.float32),
        ),
        mesh=mesh,
        compiler_params=pltpu.CompilerParams(needs_layout_passes=False),
        scratch_types=[
            pltpu.VMEM((ITP, B), jnp.int32),
            [pltpu.VMEM((B, D), jnp.float32)] * NBUF,
            pltpu.VMEM((SEG_PER_TILE, D), jnp.float32),
            pltpu.VMEM((TAIL,), jnp.int32),
            pltpu.VMEM((TAIL, D), jnp.float32),
            pltpu.VMEM((HROWS, D), jnp.float32),
            pltpu.VMEM((L,), jnp.int32),
            pltpu.VMEM((B,), jnp.int32),
            pltpu.VMEM_SHARED((NSEG, D), jnp.float32),
            pltpu.VMEM_SHARED((HROWS, D), jnp.float32),
            [pltpu.SemaphoreType.DMA] * NBUF,
            [pltpu.SemaphoreType.DMA] * NBUF,
            pltpu.SemaphoreType.DMA,
        ],
    )
    def k(feat_hbm, ids2d_hbm, ids_hbm, zsum_hbm,
          psum_hbm, pcnt_hbm,
          idx_all, rows_ring, zer_v, idx_t, rows_t, hist, idx_m, idx_blk,
          sum_sh, cnt_sh, sems_gr, sems_s, sem_m):
        c = lax.axis_index("c")
        s = lax.axis_index("s")
        wid = c * NS + s
        blk0 = wid * ITERS
        nslots = jnp.minimum(ITERS, FULL_BLOCKS - blk0)

        pltpu.sync_copy(ids2d_hbm.at[wid], idx_all)

        zvec = jnp.zeros((L,), jnp.float32)

        def zero_hist(r, carry):
            for kk in range(D // L):
                hist[r, pl.ds(kk * L, L)] = zvec
            return carry

        lax.fori_loop(0, HROWS, zero_hist, 0)
        pltpu.sync_copy(zsum_hbm, zer_v)
        pltpu.sync_copy(zer_v, sum_sh.at[pl.ds(s * SEG_PER_TILE, SEG_PER_TILE)])

        @pl.when(s == 0)
        def _():
            pltpu.sync_copy(hist, cnt_sh)

        idx_m[...] = lax.iota(jnp.int32, L)
        plsc.subcore_barrier()

        ones16 = jnp.ones((L,), jnp.float32)

        def count_slot(i, nvecs):
            for kk in range(nvecs):
                v = idx_all[i, pl.ds(kk * L, L)]
                row = lax.shift_right_logical(v, 7)
                col = lax.bitwise_and(v, 127)
                plsc.addupdate_scatter(hist, [row, col], ones16)

        def start_gather(b, kb):
            pltpu.async_copy(feat_hbm.at[pl.ds(b * B, B), :], rows_ring[kb],
                             sems_gr[kb])

        def wait_gather(b, kb):
            pltpu.make_async_copy(
                feat_hbm.at[pl.ds(b * B, B), :], rows_ring[kb],
                sems_gr[kb]).wait()

        for kb in range(NBUF):
            @pl.when(kb < nslots)
            def _(kb=kb):
                start_gather(blk0 + kb, kb)

        def block_body(j, carry):
            for kb in range(NBUF):
                i = NBUF * j + kb

                @pl.when(i < nslots)
                def _(i=i, kb=kb):
                    wait_gather(blk0 + i, kb)
                    d = pltpu.async_copy(
                        rows_ring[kb], sum_sh.at[idx_all.at[i]], sems_s[kb],
                        add=True)
                    count_slot(i, B // L)
                    d.wait()

                    @pl.when(i + NBUF < nslots)
                    def _():
                        start_gather(blk0 + i + NBUF, kb)

            return carry

        lax.fori_loop(0, (ITERS + NBUF - 1) // NBUF, block_body, 0)

        @pl.when(wid == NW - 1)
        def _():
            base = FULL_BLOCKS * B
            pltpu.sync_copy(ids_hbm.at[pl.ds(base, TAIL)], idx_t)
            pltpu.sync_copy(feat_hbm.at[pl.ds(base, TAIL), :], rows_t)
            d1 = pltpu.async_copy(rows_t, sum_sh.at[idx_t], sems_s[0],
                                  add=True)
            for kk in range(TAIL // L):
                v = idx_t[pl.ds(kk * L, L)]
                row = lax.shift_right_logical(v, 7)
                col = lax.bitwise_and(v, 127)
                plsc.addupdate_scatter(hist, [row, col], ones16)
            d1.wait()

        pltpu.async_copy(hist, cnt_sh.at[idx_m], sem_m, add=True).wait()

        plsc.subcore_barrier()

        seg0 = s * SEG_PER_TILE
        pltpu.sync_copy(sum_sh.at[pl.ds(seg0, SEG_PER_TILE)], zer_v)
        pltpu.sync_copy(zer_v, psum_hbm.at[c, pl.ds(seg0, SEG_PER_TILE), :])

        @pl.when(s == 0)
        def _():
            pltpu.sync_copy(cnt_sh, hist)
            pltpu.sync_copy(hist, pcnt_hbm.at[c])

    return k(node_feat, ids2d, seg_ids, zsum)


def _combine_body(psum_ref, pcnt_ref, out_ref):
    sums = psum_ref[0] + psum_ref[1]
    cnts = pcnt_ref[0] + pcnt_ref[1]
    out_ref[...] = sums / jnp.maximum(cnts, 1.0)


def kernel(node_feat, segment_ids):
    ids32 = segment_ids.astype(jnp.int32)
    ids2d = jnp.pad(ids32[: FULL_BLOCKS * B].reshape(FULL_BLOCKS, B),
                    ((0, NW * ITERS - FULL_BLOCKS), (0, 0)))
    ids2d = jnp.pad(ids2d.reshape(NW, ITERS, B), ((0, 0), (0, ITP - ITERS), (0, 0)))
    zsum = jnp.zeros((SEG_PER_TILE, D), jnp.float32)
    psum, pcnt = _sc_partials(node_feat, ids2d, ids32, zsum)
    pcnt_col = pcnt.reshape(NC, HROWS * D)[:, :NSEG, None]
    return pl.pallas_call(
        _combine_body,
        out_shape=jax.ShapeDtypeStruct((NSEG, D), jnp.float32),
    )(psum, pcnt_col)
